# Initial kernel scaffold; baseline (speedup 1.0000x reference)
import jax
import jax.numpy as jnp
from jax.experimental import pallas as pl


def kernel(x, f3, w_d1, b_d1, w_d2, b_d2, w_d3, b_d3, w_t1, b_t1, w_t2, b_t2, w_t3, b_t3):
    raise NotImplementedError("write your pallas kernel here")


# speedup vs baseline: 20.9859x; 20.9859x over previous
import numpy as np
import jax
import jax.numpy as jnp
from jax.experimental import pallas as pl
from jax.experimental.pallas import tpu as pltpu

IM_SZ = 448
PAD = 224
TOPN = 6
IOU_THRESH = 0.25
PART = 224
NEG = -1e30
BIG = 1e9


def _anchors_np():
    settings = (
        dict(stride=32, size=48, scale=[2 ** (1.0 / 3.0), 2 ** (2.0 / 3.0)], ar=[0.667, 1, 1.5]),
        dict(stride=64, size=96, scale=[2 ** (1.0 / 3.0), 2 ** (2.0 / 3.0)], ar=[0.667, 1, 1.5]),
        dict(stride=128, size=192, scale=[1, 2 ** (1.0 / 3.0), 2 ** (2.0 / 3.0)], ar=[0.667, 1, 1.5]),
    )
    edges = []
    for s in settings:
        stride = s['stride']
        oh = int(np.ceil(IM_SZ / stride)); ow = oh
        ostart = stride / 2.0
        oy = np.arange(ostart, ostart + stride * oh, stride).reshape(oh, 1)
        ox = np.arange(ostart, ostart + stride * ow, stride).reshape(1, ow)
        cy = np.broadcast_to(oy, (oh, ow)).astype(np.float32)
        cx = np.broadcast_to(ox, (oh, ow)).astype(np.float32)
        for sc in s['scale']:
            for ar in s['ar']:
                hh = s['size'] * sc / float(ar) ** 0.5
                ww = s['size'] * sc * float(ar) ** 0.5
                e = np.stack([cy - hh / 2.0, cx - ww / 2.0, cy + hh / 2.0, cx + ww / 2.0], -1).reshape(-1, 4)
                edges.append(e)
    return np.concatenate(edges, 0).astype(np.float32)


_ANCH_I = (_anchors_np() + PAD).astype(np.int32)
_ANCH_F = _ANCH_I.astype(np.float32)
_AREAS = (_ANCH_F[:, 2] - _ANCH_F[:, 0]) * (_ANCH_F[:, 3] - _ANCH_F[:, 1])


def _level_tab(P, side, C, g0):
    rows = P * P
    tab = np.zeros((8, C, rows), np.float32)
    tab[4] = 1.0
    tab[5] = BIG
    for c in range(C):
        for y in range(side):
            for x in range(side):
                r = y * P + x
                g = g0 + c * side * side + y * side + x
                tab[0, c, r] = _ANCH_F[g, 0]
                tab[1, c, r] = _ANCH_F[g, 1]
                tab[2, c, r] = _ANCH_F[g, 2]
                tab[3, c, r] = _ANCH_F[g, 3]
                tab[4, c, r] = _AREAS[g]
                tab[5, c, r] = g
                tab[6, c, r] = 1.0
    return tab


_TAB1 = _level_tab(16, 14, 6, 0)
_TAB2 = _level_tab(8, 7, 6, 1176)
_TAB3 = _level_tab(4, 4, 9, 1470)

_S2 = np.zeros((64, 256), np.float32)
for _y in range(7):
    for _x in range(7):
        _S2[_y * 8 + _x, 32 * _y + 2 * _x] = 1.0
_S3 = np.zeros((16, 64), np.float32)
for _y in range(4):
    for _x in range(4):
        _S3[_y * 4 + _x, 16 * _y + 2 * _x] = 1.0

_OFFS16 = [16 * dy + dx for dy in range(3) for dx in range(3)]
_OFFS8 = [8 * dy + dx for dy in range(3) for dx in range(3)]


def _rpn_nms_kernel(f3_ref, w1_ref, w2_ref, w3_ref, wt1_ref, wt2_ref, wt3_ref,
                    b1_ref, b2_ref, b3_ref, bt1_ref, bt2_ref, bt3_ref,
                    tab1_ref, tab2_ref, tab3_ref, s2m_ref, s3m_ref, out_ref):
    f3p = f3_ref[0]

    acc = None
    for i, k in enumerate(_OFFS16):
        y = jnp.dot(f3p, w1_ref[i * 2048:(i + 1) * 2048, :],
                    preferred_element_type=jnp.float32)
        part = y[k:k + 256, :]
        acc = part if acc is None else acc + part
    d1 = jax.nn.relu(acc + b1_ref[...])
    rid = jax.lax.broadcasted_iota(jnp.int32, (256, 1), 0)
    rowok = ((rid % 16) < 14) & (rid < 224)
    d1 = jnp.where(rowok, d1, 0.0)

    s1 = jnp.dot(d1, wt1_ref[...], preferred_element_type=jnp.float32) + bt1_ref[...]

    d1big = jnp.pad(d1, ((17, 23), (0, 0)))
    acc2 = None
    for i, k in enumerate(_OFFS16):
        y = jnp.dot(d1big, w2_ref[i * 128:(i + 1) * 128, :],
                    preferred_element_type=jnp.float32)
        part = y[k:k + 256, :]
        acc2 = part if acc2 is None else acc2 + part
    d2full = jax.nn.relu(acc2 + b2_ref[...])
    d2 = jnp.dot(s2m_ref[...], d2full, preferred_element_type=jnp.float32)

    s2 = jnp.dot(d2, wt2_ref[...], preferred_element_type=jnp.float32) + bt2_ref[...]

    d2big = jnp.pad(d2, ((9, 15), (0, 0)))
    acc3 = None
    for i, k in enumerate(_OFFS8):
        y = jnp.dot(d2big, w3_ref[i * 128:(i + 1) * 128, :],
                    preferred_element_type=jnp.float32)
        part = y[k:k + 64, :]
        acc3 = part if acc3 is None else acc3 + part
    d3full = jax.nn.relu(acc3 + b3_ref[...])
    d3 = jnp.dot(s3m_ref[...], d3full, preferred_element_type=jnp.float32)

    s3 = jnp.dot(d3, wt3_ref[...], preferred_element_type=jnp.float32) + bt3_ref[...]

    s1t = s1.T
    s2t = s2.T
    s3t = s3.T

    t1 = tab1_ref[...]
    t2 = tab2_ref[...]
    t3 = tab3_ref[...]
    v1 = t1[6] > 0.0
    v2 = t2[6] > 0.0
    v3 = t3[6] > 0.0
    s1m = jnp.where(v1, s1t, NEG)
    s2m = jnp.where(v2, s2t, NEG)
    s3m = jnp.where(v3, s3t, NEG)

    probs = []
    sely0 = []
    selx0 = []
    sely1 = []
    selx1 = []
    for _ in range(TOPN):
        c1 = jnp.where(v1, s1m, NEG)
        c2 = jnp.where(v2, s2m, NEG)
        c3 = jnp.where(v3, s3m, NEG)
        m = jnp.maximum(jnp.maximum(jnp.max(c1), jnp.max(c2)), jnp.max(c3))
        h1 = c1 == m
        h2 = c2 == m
        h3 = c3 == m
        gm = jnp.minimum(jnp.minimum(
            jnp.min(jnp.where(h1, t1[5], BIG)),
            jnp.min(jnp.where(h2, t2[5], BIG))),
            jnp.min(jnp.where(h3, t3[5], BIG)))
        e1 = h1 & (t1[5] == gm)
        e2 = h2 & (t2[5] == gm)
        e3 = h3 & (t3[5] == gm)

        def pick(row):
            return (jnp.sum(jnp.where(e1, t1[row], 0.0)) +
                    jnp.sum(jnp.where(e2, t2[row], 0.0)) +
                    jnp.sum(jnp.where(e3, t3[row], 0.0)))
        by0 = pick(0); bx0 = pick(1); by1 = pick(2); bx1 = pick(3)
        barea = (by1 - by0) * (bx1 - bx0)

        def supp(t, v):
            ih = jnp.maximum(jnp.minimum(t[2], by1) - jnp.maximum(t[0], by0), 0.0)
            iw = jnp.maximum(jnp.minimum(t[3], bx1) - jnp.maximum(t[1], bx0), 0.0)
            inter = ih * iw
            iou = inter / (t[4] + barea - inter)
            return v & (iou < IOU_THRESH)
        v1 = supp(t1, v1)
        v2 = supp(t2, v2)
        v3 = supp(t3, v3)

        probs.append(m)
        sely0.append(by0); selx0.append(bx0); sely1.append(by1); selx1.append(bx1)

    lane = jax.lax.broadcasted_iota(jnp.int32, (1, 128), 1)

    def row_of(vals):
        r = jnp.zeros((1, 128), jnp.float32)
        for t, v in enumerate(vals):
            r = r + jnp.where(lane == t, v, 0.0)
        return r

    out = jnp.concatenate([
        row_of(probs), row_of(sely0), row_of(selx0), row_of(sely1), row_of(selx1),
        jnp.zeros((3, 128), jnp.float32)], axis=0)
    out_ref[0] = out


def _run_rpn_nms(f3pad, w1r, w2r, w3r, wt1, wt2, wt3, b1, b2, b3, bt1, bt2, bt3):
    B = f3pad.shape[0]
    whole = lambda a: pl.BlockSpec(a.shape, lambda b: (0,) * a.ndim)
    consts = [w1r, w2r, w3r, wt1, wt2, wt3, b1, b2, b3, bt1, bt2, bt3,
              jnp.asarray(_TAB1), jnp.asarray(_TAB2), jnp.asarray(_TAB3),
              jnp.asarray(_S2), jnp.asarray(_S3)]
    return pl.pallas_call(
        _rpn_nms_kernel,
        grid=(B,),
        in_specs=[pl.BlockSpec((1, 296, 2048), lambda b: (b, 0, 0))] +
                 [whole(a) for a in consts],
        out_specs=pl.BlockSpec((1, 8, 128), lambda b: (b, 0, 0)),
        out_shape=jax.ShapeDtypeStruct((B, 8, 128), jnp.float32),
        compiler_params=pltpu.CompilerParams(
            dimension_semantics=("parallel",)),
    )(f3pad, *consts)


def _crop_kernel(boxes_sm, x_ref, out_ref):
    b = pl.program_id(0)
    p = pl.program_id(1)
    q = (b * TOPN + p) * 4
    by0 = boxes_sm[q]
    bx0 = boxes_sm[q + 1]
    by1 = boxes_sm[q + 2]
    bx1 = boxes_sm[q + 3]

    Hp = IM_SZ + 2 * PAD

    tcol = jax.lax.broadcasted_iota(jnp.int32, (PART, 1), 0).astype(jnp.float32) / (PART - 1.0)
    ys = by0 + tcol * (by1 - by0 - 1.0)
    y0f = jnp.clip(jnp.floor(ys), 0.0, Hp - 1.0)
    wy = ys - y0f
    ry0 = y0f.astype(jnp.int32) - PAD
    ry1 = jnp.minimum(ry0 + 1, Hp - 1 - PAD)
    li = jax.lax.broadcasted_iota(jnp.int32, (PART, IM_SZ), 1)
    Wy = (jnp.where(li == ry0, 1.0 - wy, 0.0) +
          jnp.where(li == ry1, wy, 0.0))

    trow = jax.lax.broadcasted_iota(jnp.int32, (1, PART), 1).astype(jnp.float32) / (PART - 1.0)
    xs = bx0 + trow * (bx1 - bx0 - 1.0)
    x0f = jnp.clip(jnp.floor(xs), 0.0, Hp - 1.0)
    wx = xs - x0f
    rx0 = x0f.astype(jnp.int32) - PAD
    rx1 = jnp.minimum(rx0 + 1, Hp - 1 - PAD)
    si = jax.lax.broadcasted_iota(jnp.int32, (IM_SZ, PART), 0)
    Wx = (jnp.where(si == rx0, 1.0 - wx, 0.0) +
          jnp.where(si == rx1, wx, 0.0))

    img = x_ref[0]
    for c in range(3):
        tmp = jnp.dot(img[c], Wx, preferred_element_type=jnp.float32)
        out_ref[0, 0, c] = jnp.dot(Wy, tmp, preferred_element_type=jnp.float32)


def _run_crop(x, boxes_flat):
    B = x.shape[0]
    grid_spec = pltpu.PrefetchScalarGridSpec(
        num_scalar_prefetch=1,
        grid=(B, TOPN),
        in_specs=[pl.BlockSpec((1, 3, IM_SZ, IM_SZ), lambda b, p, sref: (b, 0, 0, 0))],
        out_specs=pl.BlockSpec((1, 1, 3, PART, PART),
                               lambda b, p, sref: (b, p, 0, 0, 0)),
    )
    return pl.pallas_call(
        _crop_kernel,
        grid_spec=grid_spec,
        out_shape=jax.ShapeDtypeStruct((B, TOPN, 3, PART, PART), jnp.float32),
        compiler_params=pltpu.CompilerParams(
            dimension_semantics=("parallel", "arbitrary")),
    )(boxes_flat, x)


def kernel(x, f3, w_d1, b_d1, w_d2, b_d2, w_d3, b_d3, w_t1, b_t1, w_t2, b_t2, w_t3, b_t3):
    B = f3.shape[0]
    f3t = jnp.transpose(f3, (0, 2, 3, 1))
    f3p = jnp.pad(f3t, ((0, 0), (1, 1), (1, 1), (0, 0))).reshape(B, 256, 2048)
    f3pad = jnp.pad(f3p, ((0, 0), (0, 40), (0, 0)))

    w1r = jnp.transpose(w_d1, (2, 3, 1, 0)).reshape(9 * 2048, 128)
    w2r = jnp.transpose(w_d2, (2, 3, 1, 0)).reshape(9 * 128, 128)
    w3r = jnp.transpose(w_d3, (2, 3, 1, 0)).reshape(9 * 128, 128)
    wt1 = w_t1[:, :, 0, 0].T
    wt2 = w_t2[:, :, 0, 0].T
    wt3 = w_t3[:, :, 0, 0].T
    b1 = b_d1[None, :]
    b2 = b_d2[None, :]
    b3 = b_d3[None, :]
    bt1 = b_t1[None, :]
    bt2 = b_t2[None, :]
    bt3 = b_t3[None, :]

    res = _run_rpn_nms(f3pad, w1r, w2r, w3r, wt1, wt2, wt3, b1, b2, b3, bt1, bt2, bt3)
    top_prob = res[:, 0, :TOPN]
    boxes = jnp.transpose(res[:, 1:5, :TOPN], (0, 2, 1))
    part_imgs = _run_crop(x, boxes.reshape(B * TOPN * 4))
    return top_prob, part_imgs


# speedup vs baseline: 23.5372x; 1.1216x over previous
import numpy as np
import jax
import jax.numpy as jnp
from jax.experimental import pallas as pl
from jax.experimental.pallas import tpu as pltpu

IM_SZ = 448
PAD = 224
TOPN = 6
IOU_THRESH = 0.25
PART = 224
NEG = -1e30
BIG = 1e9


def _anchors_np():
    settings = (
        dict(stride=32, size=48, scale=[2 ** (1.0 / 3.0), 2 ** (2.0 / 3.0)], ar=[0.667, 1, 1.5]),
        dict(stride=64, size=96, scale=[2 ** (1.0 / 3.0), 2 ** (2.0 / 3.0)], ar=[0.667, 1, 1.5]),
        dict(stride=128, size=192, scale=[1, 2 ** (1.0 / 3.0), 2 ** (2.0 / 3.0)], ar=[0.667, 1, 1.5]),
    )
    edges = []
    for s in settings:
        stride = s['stride']
        oh = int(np.ceil(IM_SZ / stride)); ow = oh
        ostart = stride / 2.0
        oy = np.arange(ostart, ostart + stride * oh, stride).reshape(oh, 1)
        ox = np.arange(ostart, ostart + stride * ow, stride).reshape(1, ow)
        cy = np.broadcast_to(oy, (oh, ow)).astype(np.float32)
        cx = np.broadcast_to(ox, (oh, ow)).astype(np.float32)
        for sc in s['scale']:
            for ar in s['ar']:
                hh = s['size'] * sc / float(ar) ** 0.5
                ww = s['size'] * sc * float(ar) ** 0.5
                e = np.stack([cy - hh / 2.0, cx - ww / 2.0, cy + hh / 2.0, cx + ww / 2.0], -1).reshape(-1, 4)
                edges.append(e)
    return np.concatenate(edges, 0).astype(np.float32)


_ANCH_I = (_anchors_np() + PAD).astype(np.int32)
_ANCH_F = _ANCH_I.astype(np.float32)
_AREAS = (_ANCH_F[:, 2] - _ANCH_F[:, 0]) * (_ANCH_F[:, 3] - _ANCH_F[:, 1])


def _level_tab(side, C, g0):
    rows = side * side
    tab = np.zeros((8, C, rows), np.float32)
    for c in range(C):
        for y in range(side):
            for x in range(side):
                r = y * side + x
                g = g0 + c * rows + r
                tab[0, c, r] = _ANCH_F[g, 0]
                tab[1, c, r] = _ANCH_F[g, 1]
                tab[2, c, r] = _ANCH_F[g, 2]
                tab[3, c, r] = _ANCH_F[g, 3]
                tab[4, c, r] = _AREAS[g]
                tab[5, c, r] = g
    return tab


_TAB1 = _level_tab(14, 6, 0)
_TAB2 = _level_tab(7, 6, 1176)
_TAB3 = _level_tab(4, 9, 1470)

_S2SEL = np.zeros((196, 64), np.float32)
for _y in range(7):
    for _x in range(7):
        _S2SEL[14 * (2 * _y) + 2 * _x, _y * 7 + _x] = 1.0
_S3SEL = np.zeros((49, 16), np.float32)
for _y in range(4):
    for _x in range(4):
        _S3SEL[7 * (2 * _y) + 2 * _x, _y * 4 + _x] = 1.0


def _conv_t(X, w_ref, L, P, off0):
    col = jax.lax.broadcasted_iota(jnp.int32, (1, L), 1)
    acc = None
    for dy in range(3):
        for dx in range(3):
            i = dy * 3 + dx
            y = jnp.dot(w_ref[i * 128:(i + 1) * 128, :], X,
                        preferred_element_type=jnp.float32)
            s = P * dy + dx - off0
            yp = jnp.pad(y, ((0, 0), (off0, off0)))
            part = yp[:, off0 + s: off0 + s + L]
            if dx == 0:
                part = jnp.where((col % P) != 0, part, 0.0)
            elif dx == 2:
                part = jnp.where((col % P) != (P - 1), part, 0.0)
            acc = part if acc is None else acc + part
    return acc


def _rpn_nms_kernel(f3_ref, w1_ref, w2_ref, w3_ref, wt1_ref, wt2_ref, wt3_ref,
                    b1_ref, b2_ref, b3_ref, bt1_ref, bt2_ref, bt3_ref,
                    tab1_ref, tab2_ref, tab3_ref, s2s_ref, s3s_ref, out_ref):
    X = f3_ref[0]
    o1 = _conv_t(X, w1_ref, 196, 14, 15)
    d1 = jax.nn.relu(o1 + b1_ref[...])

    s1 = jnp.dot(wt1_ref[...], d1, preferred_element_type=jnp.float32) + bt1_ref[...]

    o2 = _conv_t(d1, w2_ref, 196, 14, 15)
    d2 = jnp.dot(jax.nn.relu(o2 + b2_ref[...]), s2s_ref[...],
                 preferred_element_type=jnp.float32)[:, :49]

    s2 = jnp.dot(wt2_ref[...], d2, preferred_element_type=jnp.float32) + bt2_ref[...]

    o3 = _conv_t(d2, w3_ref, 49, 7, 8)
    d3 = jnp.dot(jax.nn.relu(o3 + b3_ref[...]), s3s_ref[...],
                 preferred_element_type=jnp.float32)

    s3 = jnp.dot(wt3_ref[...], d3, preferred_element_type=jnp.float32) + bt3_ref[...]

    t1 = tab1_ref[...]
    t2 = tab2_ref[...]
    t3 = tab3_ref[...]
    v1 = jnp.ones(s1.shape, dtype=jnp.bool_)
    v2 = jnp.ones(s2.shape, dtype=jnp.bool_)
    v3 = jnp.ones(s3.shape, dtype=jnp.bool_)

    probs = []
    sely0 = []
    selx0 = []
    sely1 = []
    selx1 = []
    for _ in range(TOPN):
        c1 = jnp.where(v1, s1, NEG)
        c2 = jnp.where(v2, s2, NEG)
        c3 = jnp.where(v3, s3, NEG)
        m = jnp.maximum(jnp.maximum(jnp.max(c1), jnp.max(c2)), jnp.max(c3))
        h1 = c1 == m
        h2 = c2 == m
        h3 = c3 == m
        gm = jnp.minimum(jnp.minimum(
            jnp.min(jnp.where(h1, t1[5], BIG)),
            jnp.min(jnp.where(h2, t2[5], BIG))),
            jnp.min(jnp.where(h3, t3[5], BIG)))
        e1 = h1 & (t1[5] == gm)
        e2 = h2 & (t2[5] == gm)
        e3 = h3 & (t3[5] == gm)

        def pick(row):
            return (jnp.sum(jnp.where(e1, t1[row], 0.0)) +
                    jnp.sum(jnp.where(e2, t2[row], 0.0)) +
                    jnp.sum(jnp.where(e3, t3[row], 0.0)))
        by0 = pick(0); bx0 = pick(1); by1 = pick(2); bx1 = pick(3)
        barea = (by1 - by0) * (bx1 - bx0)

        def supp(t, v):
            ih = jnp.maximum(jnp.minimum(t[2], by1) - jnp.maximum(t[0], by0), 0.0)
            iw = jnp.maximum(jnp.minimum(t[3], bx1) - jnp.maximum(t[1], bx0), 0.0)
            inter = ih * iw
            iou = inter / (t[4] + barea - inter)
            return v & (iou < IOU_THRESH)
        v1 = supp(t1, v1)
        v2 = supp(t2, v2)
        v3 = supp(t3, v3)

        probs.append(m)
        sely0.append(by0); selx0.append(bx0); sely1.append(by1); selx1.append(bx1)

    lane = jax.lax.broadcasted_iota(jnp.int32, (1, 128), 1)

    def row_of(vals):
        r = jnp.zeros((1, 128), jnp.float32)
        for t, v in enumerate(vals):
            r = r + jnp.where(lane == t, v, 0.0)
        return r

    out = jnp.concatenate([
        row_of(probs), row_of(sely0), row_of(selx0), row_of(sely1), row_of(selx1),
        jnp.zeros((3, 128), jnp.float32)], axis=0)
    out_ref[0] = out


def _run_rpn_nms(f3r, w1r, w2r, w3r, wt1, wt2, wt3, b1, b2, b3, bt1, bt2, bt3):
    B = f3r.shape[0]
    whole = lambda a: pl.BlockSpec(a.shape, lambda b: (0,) * a.ndim)
    consts = [w1r, w2r, w3r, wt1, wt2, wt3, b1, b2, b3, bt1, bt2, bt3,
              jnp.asarray(_TAB1), jnp.asarray(_TAB2), jnp.asarray(_TAB3),
              jnp.asarray(_S2SEL), jnp.asarray(_S3SEL)]
    return pl.pallas_call(
        _rpn_nms_kernel,
        grid=(B,),
        in_specs=[pl.BlockSpec((1, 2048, 196), lambda b: (b, 0, 0))] +
                 [whole(a) for a in consts],
        out_specs=pl.BlockSpec((1, 8, 128), lambda b: (b, 0, 0)),
        out_shape=jax.ShapeDtypeStruct((B, 8, 128), jnp.float32),
        compiler_params=pltpu.CompilerParams(
            dimension_semantics=("parallel",)),
    )(f3r, *consts)


def _crop_kernel(boxes_sm, x_ref, out_ref):
    b = pl.program_id(0)
    p = pl.program_id(1)
    q = (b * TOPN + p) * 4
    by0 = boxes_sm[q]
    bx0 = boxes_sm[q + 1]
    by1 = boxes_sm[q + 2]
    bx1 = boxes_sm[q + 3]

    Hp = IM_SZ + 2 * PAD

    tcol = jax.lax.broadcasted_iota(jnp.int32, (PART, 1), 0).astype(jnp.float32) / (PART - 1.0)
    ys = by0 + tcol * (by1 - by0 - 1.0)
    y0f = jnp.clip(jnp.floor(ys), 0.0, Hp - 1.0)
    wy = ys - y0f
    ry0 = y0f.astype(jnp.int32) - PAD
    ry1 = jnp.minimum(ry0 + 1, Hp - 1 - PAD)
    li = jax.lax.broadcasted_iota(jnp.int32, (PART, IM_SZ), 1)
    Wy = (jnp.where(li == ry0, 1.0 - wy, 0.0) +
          jnp.where(li == ry1, wy, 0.0))

    trow = jax.lax.broadcasted_iota(jnp.int32, (1, PART), 1).astype(jnp.float32) / (PART - 1.0)
    xs = bx0 + trow * (bx1 - bx0 - 1.0)
    x0f = jnp.clip(jnp.floor(xs), 0.0, Hp - 1.0)
    wx = xs - x0f
    rx0 = x0f.astype(jnp.int32) - PAD
    rx1 = jnp.minimum(rx0 + 1, Hp - 1 - PAD)
    si = jax.lax.broadcasted_iota(jnp.int32, (IM_SZ, PART), 0)
    Wx = (jnp.where(si == rx0, 1.0 - wx, 0.0) +
          jnp.where(si == rx1, wx, 0.0))

    img = x_ref[0]
    for c in range(3):
        tmp = jnp.dot(img[c], Wx, preferred_element_type=jnp.float32)
        out_ref[0, 0, c] = jnp.dot(Wy, tmp, preferred_element_type=jnp.float32)


def _run_crop(x, boxes_flat):
    B = x.shape[0]
    grid_spec = pltpu.PrefetchScalarGridSpec(
        num_scalar_prefetch=1,
        grid=(B, TOPN),
        in_specs=[pl.BlockSpec((1, 3, IM_SZ, IM_SZ), lambda b, p, sref: (b, 0, 0, 0))],
        out_specs=pl.BlockSpec((1, 1, 3, PART, PART),
                               lambda b, p, sref: (b, p, 0, 0, 0)),
    )
    return pl.pallas_call(
        _crop_kernel,
        grid_spec=grid_spec,
        out_shape=jax.ShapeDtypeStruct((B, TOPN, 3, PART, PART), jnp.float32),
        compiler_params=pltpu.CompilerParams(
            dimension_semantics=("parallel", "arbitrary")),
    )(boxes_flat, x)


def kernel(x, f3, w_d1, b_d1, w_d2, b_d2, w_d3, b_d3, w_t1, b_t1, w_t2, b_t2, w_t3, b_t3):
    B = f3.shape[0]
    f3r = f3.reshape(B, 2048, 196)

    w1r = jnp.transpose(w_d1, (2, 3, 0, 1)).reshape(9 * 128, 2048)
    w2r = jnp.transpose(w_d2, (2, 3, 0, 1)).reshape(9 * 128, 128)
    w3r = jnp.transpose(w_d3, (2, 3, 0, 1)).reshape(9 * 128, 128)
    wt1 = w_t1[:, :, 0, 0]
    wt2 = w_t2[:, :, 0, 0]
    wt3 = w_t3[:, :, 0, 0]
    b1 = b_d1[:, None]
    b2 = b_d2[:, None]
    b3 = b_d3[:, None]
    bt1 = b_t1[:, None]
    bt2 = b_t2[:, None]
    bt3 = b_t3[:, None]

    res = _run_rpn_nms(f3r, w1r, w2r, w3r, wt1, wt2, wt3, b1, b2, b3, bt1, bt2, bt3)
    top_prob = res[:, 0, :TOPN]
    boxes = jnp.transpose(res[:, 1:5, :TOPN], (0, 2, 1))
    part_imgs = _run_crop(x, boxes.reshape(B * TOPN * 4))
    return top_prob, part_imgs
